# baseline (speedup 1.0000x reference)
import functools

import jax
import jax.numpy as jnp
from jax import lax
from jax.experimental import pallas as pl
from jax.experimental.pallas import tpu as pltpu
from jax.experimental.pallas import tpu_sc as plsc

NN = 10000
CC = 128
EE = 320000
NC = 2
NS = 16
NW = NC * NS
CHUNK = 80
KT = 256
K0 = 176
K1 = 80
IBLK = 8
NRING = 3
E_PAD = NS * KT * CHUNK
IDX_ROWS = E_PAD // CHUNK
CHUNKC = 128
IDX_ROWS_C = E_PAD // CHUNKC
CPWC = IDX_ROWS_C // NW
N_ACC = 10112
STRIPE = N_ACC // NS

_MESH = dict(core_axis_name="c", subcore_axis_name="s", num_cores=NC,
             num_subcores=NS)


def _agg_body(t1, t2, t3, t4, t5, t6, srcm, dstm, zacc,
              acc_out,
              acc_sh, src_v, dst_v, r0, r1, r2, s0, s1, s2):
    c = lax.axis_index("c")
    s = lax.axis_index("s")
    pltpu.sync_copy(zacc.at[pl.ds(s * STRIPE, STRIPE)],
                    acc_sh.at[pl.ds(s * STRIPE, STRIPE)])
    plsc.subcore_barrier()

    base0 = s * KT + jnp.where(c == 0, 0, K0)
    rows = (r0, r1, r2)
    sems = (s0, s1, s2)

    def run(tabs, my_nblk):
        def g(b, j):
            return pltpu.make_async_copy(tabs[b].at[src_v.at[j]], rows[b],
                                         sems[b])

        def outer(bi, carry):
            base = base0 + bi * IBLK
            pltpu.sync_copy(srcm.at[pl.ds(base, IBLK)], src_v)
            pltpu.sync_copy(dstm.at[pl.ds(base, IBLK)], dst_v)
            for j in range(NRING):
                g(j, j).start()
            for j in range(IBLK):
                b = j % NRING
                g(b, j).wait()
                pltpu.sync_copy(rows[b], acc_sh.at[dst_v.at[j]], add=True)
                if j + NRING < IBLK:
                    g(b, j + NRING).start()
            return carry

        lax.fori_loop(0, my_nblk, outer, 0)

    @pl.when(c == 0)
    def _():
        run((t1, t3, t5), K0 // IBLK)

    @pl.when(c != 0)
    def _():
        run((t2, t4, t6), K1 // IBLK)

    plsc.subcore_barrier()
    pltpu.sync_copy(acc_sh.at[pl.ds(s * STRIPE, STRIPE)],
                    acc_out.at[c, pl.ds(s * STRIPE, STRIPE)])


def _cnt_body(dstm, zcnt, ones_hbm,
              cnt_out,
              cnt_sh, dst_v, ones_v):
    c = lax.axis_index("c")
    s = lax.axis_index("s")
    w = s * NC + c
    pltpu.sync_copy(zcnt.at[pl.ds(s * STRIPE, STRIPE)],
                    cnt_sh.at[pl.ds(s * STRIPE, STRIPE)])
    pltpu.sync_copy(ones_hbm, ones_v)
    plsc.subcore_barrier()

    def outer(bi, carry):
        base = w * CPWC + bi * IBLK
        pltpu.sync_copy(dstm.at[pl.ds(base, IBLK)], dst_v)

        def inner(j, c2):
            pltpu.sync_copy(ones_v, cnt_sh.at[dst_v.at[j]], add=True)
            return c2

        lax.fori_loop(0, IBLK, inner, 0)
        return carry

    lax.fori_loop(0, CPWC // IBLK, outer, 0)
    plsc.subcore_barrier()
    pltpu.sync_copy(cnt_sh.at[pl.ds(s * STRIPE, STRIPE)],
                    cnt_out.at[c, pl.ds(s * STRIPE, STRIPE)])


_sc_agg = pl.kernel(
    _agg_body,
    out_type=jax.ShapeDtypeStruct((NC, N_ACC, CC), jnp.float32),
    mesh=plsc.VectorSubcoreMesh(**_MESH),
    scratch_types=[
        pltpu.VMEM_SHARED((N_ACC, CC), jnp.float32),
        pltpu.VMEM((IBLK, CHUNK), jnp.int32),
        pltpu.VMEM((IBLK, CHUNK), jnp.int32),
        pltpu.VMEM((CHUNK, CC), jnp.float32),
        pltpu.VMEM((CHUNK, CC), jnp.float32),
        pltpu.VMEM((CHUNK, CC), jnp.float32),
        pltpu.SemaphoreType.DMA,
        pltpu.SemaphoreType.DMA,
        pltpu.SemaphoreType.DMA,
    ],
)

_sc_cnt = pl.kernel(
    _cnt_body,
    out_type=jax.ShapeDtypeStruct((NC, N_ACC, CC), jnp.float32),
    mesh=plsc.VectorSubcoreMesh(**_MESH),
    scratch_types=[
        pltpu.VMEM_SHARED((N_ACC, CC), jnp.float32),
        pltpu.VMEM((IBLK, CHUNKC), jnp.int32),
        pltpu.VMEM((CHUNKC, CC), jnp.float32),
    ],
)


def _tc_layer_body(relu, ncopies, acc_ref, cnt_ref, x_ref, wl_ref, wr_ref,
                   b_ref, *o_refs):
    acc = acc_ref[0] + acc_ref[1]
    cnt = cnt_ref[0] + cnt_ref[1]
    denom = jnp.maximum(cnt[:, 0:1], 1.0)
    agg = acc / denom
    h = (jnp.dot(agg, wl_ref[...], preferred_element_type=jnp.float32)
         + jnp.dot(x_ref[...], wr_ref[...], preferred_element_type=jnp.float32)
         + b_ref[...])
    h = jnp.maximum(h, 0.0) if relu else h
    for o_ref in o_refs:
        o_ref[...] = h


def _tc_layer(relu, ncopies, acc, cnt, x, wl, wr, b):
    blk = 1000
    grid = (NN // blk,)
    out = pl.pallas_call(
        functools.partial(_tc_layer_body, relu, ncopies),
        grid=grid,
        in_specs=[
            pl.BlockSpec((NC, blk, CC), lambda i: (0, i, 0)),
            pl.BlockSpec((NC, blk, CC), lambda i: (0, i, 0)),
            pl.BlockSpec((blk, CC), lambda i: (i, 0)),
            pl.BlockSpec((CC, CC), lambda i: (0, 0)),
            pl.BlockSpec((CC, CC), lambda i: (0, 0)),
            pl.BlockSpec((1, CC), lambda i: (0, 0)),
        ],
        out_specs=[pl.BlockSpec((blk, CC), lambda i: (i, 0))] * ncopies,
        out_shape=[jax.ShapeDtypeStruct((NN, CC), jnp.float32)] * ncopies,
    )(acc, cnt, x, wl, wr, b)
    return out


def kernel(x, edge_index, W1_l, W1_r, b1, W2_l, W2_r, b2):
    src = edge_index[0]
    dst = edge_index[1]
    pad = E_PAD - EE
    srcm = jnp.concatenate([src, jnp.zeros((pad,), jnp.int32)]).reshape(
        IDX_ROWS, CHUNK)
    dst_pad = jnp.concatenate([dst, jnp.full((pad,), NN, jnp.int32)])
    dstm = dst_pad.reshape(IDX_ROWS, CHUNK)
    dstmc = dst_pad.reshape(IDX_ROWS_C, CHUNKC)
    zacc = jnp.zeros((N_ACC, CC), jnp.float32)
    ones = jnp.ones((CHUNKC, CC), jnp.float32)

    zs = lax.optimization_barrier(tuple(jnp.float32(0.0) for _ in range(5)))
    xs = tuple(x + z for z in zs)
    cnt = _sc_cnt(dstmc, zacc, ones)
    acc1 = _sc_agg(x, *xs, srcm, dstm, zacc)
    hs = _tc_layer(True, 6, acc1, cnt, x, W1_l, W1_r, b1.reshape(1, CC))
    acc2 = _sc_agg(*hs, srcm, dstm, zacc)
    out, = _tc_layer(False, 1, acc2, cnt, hs[0], W2_l, W2_r, b2.reshape(1, CC))
    return out


# speedup vs baseline: 1.0266x; 1.0266x over previous
import functools

import jax
import jax.numpy as jnp
from jax import lax
from jax.experimental import pallas as pl
from jax.experimental.pallas import tpu as pltpu
from jax.experimental.pallas import tpu_sc as plsc

NN = 10000
CC = 128
EE = 320000
NC = 2
NS = 16
NW = NC * NS
CHUNK = 80
KT = 256
K0 = 176
K1 = 80
IBLK = 8
NRING = 3
E_PAD = NS * KT * CHUNK
IDX_ROWS = E_PAD // CHUNK
CHUNKC = 128
IDX_ROWS_C = E_PAD // CHUNKC
CPWC = IDX_ROWS_C // NW
N_ACC = 10112
STRIPE = N_ACC // NS

_MESH = dict(core_axis_name="c", subcore_axis_name="s", num_cores=NC,
             num_subcores=NS)


def _agg_body(t1, t2, t3, t4, t5, t6, srcm, dstm, zacc,
              acc_out,
              acc_sh, src_v, dst_v, r0, r1, r2, s0, s1, s2):
    c = lax.axis_index("c")
    s = lax.axis_index("s")
    pltpu.sync_copy(zacc.at[pl.ds(s * STRIPE, STRIPE)],
                    acc_sh.at[pl.ds(s * STRIPE, STRIPE)])
    plsc.subcore_barrier()

    base0 = s * KT + jnp.where(c == 0, 0, K0)
    rows = (r0, r1, r2)
    sems = (s0, s1, s2)

    def run(tabs, my_nblk):
        def g(b, j):
            return pltpu.make_async_copy(tabs[b].at[src_v.at[j]], rows[b],
                                         sems[b])

        def outer(bi, carry):
            base = base0 + bi * IBLK
            pltpu.sync_copy(srcm.at[pl.ds(base, IBLK)], src_v)
            pltpu.sync_copy(dstm.at[pl.ds(base, IBLK)], dst_v)
            for j in range(NRING):
                g(j, j).start()
            for j in range(IBLK):
                b = j % NRING
                g(b, j).wait()
                pltpu.sync_copy(rows[b], acc_sh.at[dst_v.at[j]], add=True)
                if j + NRING < IBLK:
                    g(b, j + NRING).start()
            return carry

        lax.fori_loop(0, my_nblk, outer, 0)

    @pl.when(c == 0)
    def _():
        run((t1, t3, t5), K0 // IBLK)

    @pl.when(c != 0)
    def _():
        run((t2, t4, t6), K1 // IBLK)

    plsc.subcore_barrier()
    pltpu.sync_copy(acc_sh.at[pl.ds(s * STRIPE, STRIPE)],
                    acc_out.at[c, pl.ds(s * STRIPE, STRIPE)])


def _cnt_body(dstm, zcnt, ones_hbm,
              cnt_out,
              cnt_sh, dst_v, ones_v):
    c = lax.axis_index("c")
    s = lax.axis_index("s")
    w = s * NC + c
    pltpu.sync_copy(zcnt.at[pl.ds(s * STRIPE, STRIPE)],
                    cnt_sh.at[pl.ds(s * STRIPE, STRIPE)])
    pltpu.sync_copy(ones_hbm, ones_v)
    plsc.subcore_barrier()

    def outer(bi, carry):
        base = w * CPWC + bi * IBLK
        pltpu.sync_copy(dstm.at[pl.ds(base, IBLK)], dst_v)

        def inner(j, c2):
            pltpu.sync_copy(ones_v, cnt_sh.at[dst_v.at[j]], add=True)
            return c2

        lax.fori_loop(0, IBLK, inner, 0)
        return carry

    lax.fori_loop(0, CPWC // IBLK, outer, 0)
    plsc.subcore_barrier()
    pltpu.sync_copy(cnt_sh.at[pl.ds(s * STRIPE, STRIPE)],
                    cnt_out.at[c, pl.ds(s * STRIPE, STRIPE)])


_sc_agg = pl.kernel(
    _agg_body,
    out_type=jax.ShapeDtypeStruct((NC, N_ACC, CC), jnp.float32),
    mesh=plsc.VectorSubcoreMesh(**_MESH),
    scratch_types=[
        pltpu.VMEM_SHARED((N_ACC, CC), jnp.float32),
        pltpu.VMEM((IBLK, CHUNK), jnp.int32),
        pltpu.VMEM((IBLK, CHUNK), jnp.int32),
        pltpu.VMEM((CHUNK, CC), jnp.float32),
        pltpu.VMEM((CHUNK, CC), jnp.float32),
        pltpu.VMEM((CHUNK, CC), jnp.float32),
        pltpu.SemaphoreType.DMA,
        pltpu.SemaphoreType.DMA,
        pltpu.SemaphoreType.DMA,
    ],
)

_sc_cnt = pl.kernel(
    _cnt_body,
    out_type=jax.ShapeDtypeStruct((NC, N_ACC, CC), jnp.float32),
    mesh=plsc.VectorSubcoreMesh(**_MESH),
    scratch_types=[
        pltpu.VMEM_SHARED((N_ACC, CC), jnp.float32),
        pltpu.VMEM((IBLK, CHUNKC), jnp.int32),
        pltpu.VMEM((CHUNKC, CC), jnp.float32),
    ],
)


def _tc_layer_body(relu, ncopies, acc_ref, cnt_ref, x_ref, wl_ref, wr_ref,
                   b_ref, *o_refs):
    acc = acc_ref[0] + acc_ref[1]
    cnt = cnt_ref[0] + cnt_ref[1]
    denom = jnp.maximum(cnt[:, 0:1], 1.0)
    agg = acc / denom
    h = (jnp.dot(agg, wl_ref[...], preferred_element_type=jnp.float32)
         + jnp.dot(x_ref[...], wr_ref[...], preferred_element_type=jnp.float32)
         + b_ref[...])
    h = jnp.maximum(h, 0.0) if relu else h
    for o_ref in o_refs:
        o_ref[...] = h


def _tc_layer(relu, ncopies, acc, cnt, x, wl, wr, b):
    blk = 1000
    grid = (NN // blk,)
    out = pl.pallas_call(
        functools.partial(_tc_layer_body, relu, ncopies),
        grid=grid,
        in_specs=[
            pl.BlockSpec((NC, blk, CC), lambda i: (0, i, 0)),
            pl.BlockSpec((NC, blk, CC), lambda i: (0, i, 0)),
            pl.BlockSpec((blk, CC), lambda i: (i, 0)),
            pl.BlockSpec((CC, CC), lambda i: (0, 0)),
            pl.BlockSpec((CC, CC), lambda i: (0, 0)),
            pl.BlockSpec((1, CC), lambda i: (0, 0)),
        ],
        out_specs=[pl.BlockSpec((blk, CC), lambda i: (i, 0))] * ncopies,
        out_shape=[jax.ShapeDtypeStruct((NN, CC), jnp.float32)] * ncopies,
    )(acc, cnt, x, wl, wr, b)
    return out


def kernel(x, edge_index, W1_l, W1_r, b1, W2_l, W2_r, b2):
    src = edge_index[0]
    dst = edge_index[1]
    pad = E_PAD - EE
    srcm = jnp.concatenate([src, jnp.zeros((pad,), jnp.int32)]).reshape(
        IDX_ROWS, CHUNK)
    dst_pad = jnp.concatenate([dst, jnp.full((pad,), NN, jnp.int32)])
    dstm = dst_pad.reshape(IDX_ROWS, CHUNK)
    dstmc = dst_pad.reshape(IDX_ROWS_C, CHUNKC)
    zacc = jnp.zeros((N_ACC, CC), jnp.float32)
    ones = jnp.ones((CHUNKC, CC), jnp.float32)

    zs = lax.optimization_barrier(tuple(jnp.float32(0.0) for _ in range(5)))
    xs = tuple(x + z for z in zs)
    cnt = _sc_cnt(dstmc, zacc, ones)
    acc1 = _sc_agg(x, *xs, srcm, dstm, zacc)
    h, = _tc_layer(True, 1, acc1, cnt, x, W1_l, W1_r, b1.reshape(1, CC))
    hs = tuple(h + z for z in zs)
    acc2 = _sc_agg(h, *hs, srcm, dstm, zacc)
    out, = _tc_layer(False, 1, acc2, cnt, h, W2_l, W2_r, b2.reshape(1, CC))
    return out


# speedup vs baseline: 1.0285x; 1.0019x over previous
import functools

import jax
import jax.numpy as jnp
from jax import lax
from jax.experimental import pallas as pl
from jax.experimental.pallas import tpu as pltpu
from jax.experimental.pallas import tpu_sc as plsc

NN = 10000
CC = 128
EE = 320000
NC = 2
NS = 16
NW = NC * NS
CHUNK = 80
KT = 256
K0 = 168
K1 = 88
IBLK = 8
NRING = 3
E_PAD = NS * KT * CHUNK
IDX_ROWS = E_PAD // CHUNK
CHUNKC = 128
IDX_ROWS_C = E_PAD // CHUNKC
CPWC = IDX_ROWS_C // NW
N_ACC = 10112
STRIPE = N_ACC // NS

_MESH = dict(core_axis_name="c", subcore_axis_name="s", num_cores=NC,
             num_subcores=NS)


def _agg_body(t1, t2, t3, t4, t5, t6, srcm, dstm, zacc,
              acc_out,
              acc_sh, src_v, dst_v, r0, r1, r2, s0, s1, s2):
    c = lax.axis_index("c")
    s = lax.axis_index("s")
    pltpu.sync_copy(zacc.at[pl.ds(s * STRIPE, STRIPE)],
                    acc_sh.at[pl.ds(s * STRIPE, STRIPE)])
    plsc.subcore_barrier()

    base0 = s * KT + jnp.where(c == 0, 0, K0)
    rows = (r0, r1, r2)
    sems = (s0, s1, s2)

    def run(tabs, my_nblk):
        def g(b, j):
            return pltpu.make_async_copy(tabs[b].at[src_v.at[j]], rows[b],
                                         sems[b])

        def outer(bi, carry):
            base = base0 + bi * IBLK
            pltpu.sync_copy(srcm.at[pl.ds(base, IBLK)], src_v)
            pltpu.sync_copy(dstm.at[pl.ds(base, IBLK)], dst_v)
            for j in range(NRING):
                g(j, j).start()
            for j in range(IBLK):
                b = j % NRING
                g(b, j).wait()
                pltpu.sync_copy(rows[b], acc_sh.at[dst_v.at[j]], add=True)
                if j + NRING < IBLK:
                    g(b, j + NRING).start()
            return carry

        lax.fori_loop(0, my_nblk, outer, 0)

    @pl.when(c == 0)
    def _():
        run((t1, t3, t5), K0 // IBLK)

    @pl.when(c != 0)
    def _():
        run((t2, t4, t6), K1 // IBLK)

    plsc.subcore_barrier()
    pltpu.sync_copy(acc_sh.at[pl.ds(s * STRIPE, STRIPE)],
                    acc_out.at[c, pl.ds(s * STRIPE, STRIPE)])


def _cnt_body(dstm, zcnt, ones_hbm,
              cnt_out,
              cnt_sh, dst_v, ones_v):
    c = lax.axis_index("c")
    s = lax.axis_index("s")
    w = s * NC + c
    pltpu.sync_copy(zcnt.at[pl.ds(s * STRIPE, STRIPE)],
                    cnt_sh.at[pl.ds(s * STRIPE, STRIPE)])
    pltpu.sync_copy(ones_hbm, ones_v)
    plsc.subcore_barrier()

    def outer(bi, carry):
        base = w * CPWC + bi * IBLK
        pltpu.sync_copy(dstm.at[pl.ds(base, IBLK)], dst_v)

        def inner(j, c2):
            pltpu.sync_copy(ones_v, cnt_sh.at[dst_v.at[j]], add=True)
            return c2

        lax.fori_loop(0, IBLK, inner, 0)
        return carry

    lax.fori_loop(0, CPWC // IBLK, outer, 0)
    plsc.subcore_barrier()
    pltpu.sync_copy(cnt_sh.at[pl.ds(s * STRIPE, STRIPE)],
                    cnt_out.at[c, pl.ds(s * STRIPE, STRIPE)])


_sc_agg = pl.kernel(
    _agg_body,
    out_type=jax.ShapeDtypeStruct((NC, N_ACC, CC), jnp.float32),
    mesh=plsc.VectorSubcoreMesh(**_MESH),
    scratch_types=[
        pltpu.VMEM_SHARED((N_ACC, CC), jnp.float32),
        pltpu.VMEM((IBLK, CHUNK), jnp.int32),
        pltpu.VMEM((IBLK, CHUNK), jnp.int32),
        pltpu.VMEM((CHUNK, CC), jnp.float32),
        pltpu.VMEM((CHUNK, CC), jnp.float32),
        pltpu.VMEM((CHUNK, CC), jnp.float32),
        pltpu.SemaphoreType.DMA,
        pltpu.SemaphoreType.DMA,
        pltpu.SemaphoreType.DMA,
    ],
)

_sc_cnt = pl.kernel(
    _cnt_body,
    out_type=jax.ShapeDtypeStruct((NC, N_ACC, CC), jnp.float32),
    mesh=plsc.VectorSubcoreMesh(**_MESH),
    scratch_types=[
        pltpu.VMEM_SHARED((N_ACC, CC), jnp.float32),
        pltpu.VMEM((IBLK, CHUNKC), jnp.int32),
        pltpu.VMEM((CHUNKC, CC), jnp.float32),
    ],
)


def _tc_layer_body(relu, ncopies, acc_ref, cnt_ref, x_ref, wl_ref, wr_ref,
                   b_ref, *o_refs):
    acc = acc_ref[0] + acc_ref[1]
    cnt = cnt_ref[0] + cnt_ref[1]
    denom = jnp.maximum(cnt[:, 0:1], 1.0)
    agg = acc / denom
    h = (jnp.dot(agg, wl_ref[...], preferred_element_type=jnp.float32)
         + jnp.dot(x_ref[...], wr_ref[...], preferred_element_type=jnp.float32)
         + b_ref[...])
    h = jnp.maximum(h, 0.0) if relu else h
    for o_ref in o_refs:
        o_ref[...] = h


def _tc_layer(relu, ncopies, acc, cnt, x, wl, wr, b):
    blk = 1000
    grid = (NN // blk,)
    out = pl.pallas_call(
        functools.partial(_tc_layer_body, relu, ncopies),
        grid=grid,
        in_specs=[
            pl.BlockSpec((NC, blk, CC), lambda i: (0, i, 0)),
            pl.BlockSpec((NC, blk, CC), lambda i: (0, i, 0)),
            pl.BlockSpec((blk, CC), lambda i: (i, 0)),
            pl.BlockSpec((CC, CC), lambda i: (0, 0)),
            pl.BlockSpec((CC, CC), lambda i: (0, 0)),
            pl.BlockSpec((1, CC), lambda i: (0, 0)),
        ],
        out_specs=[pl.BlockSpec((blk, CC), lambda i: (i, 0))] * ncopies,
        out_shape=[jax.ShapeDtypeStruct((NN, CC), jnp.float32)] * ncopies,
    )(acc, cnt, x, wl, wr, b)
    return out


def kernel(x, edge_index, W1_l, W1_r, b1, W2_l, W2_r, b2):
    src = edge_index[0]
    dst = edge_index[1]
    pad = E_PAD - EE
    srcm = jnp.concatenate([src, jnp.zeros((pad,), jnp.int32)]).reshape(
        IDX_ROWS, CHUNK)
    dst_pad = jnp.concatenate([dst, jnp.full((pad,), NN, jnp.int32)])
    dstm = dst_pad.reshape(IDX_ROWS, CHUNK)
    dstmc = dst_pad.reshape(IDX_ROWS_C, CHUNKC)
    zacc = jnp.zeros((N_ACC, CC), jnp.float32)
    ones = jnp.ones((CHUNKC, CC), jnp.float32)

    zs = lax.optimization_barrier(tuple(jnp.float32(0.0) for _ in range(5)))
    xs = tuple(x + z for z in zs)
    cnt = _sc_cnt(dstmc, zacc, ones)
    acc1 = _sc_agg(x, *xs, srcm, dstm, zacc)
    h, = _tc_layer(True, 1, acc1, cnt, x, W1_l, W1_r, b1.reshape(1, CC))
    hs = tuple(h + z for z in zs)
    acc2 = _sc_agg(h, *hs, srcm, dstm, zacc)
    out, = _tc_layer(False, 1, acc2, cnt, h, W2_l, W2_r, b2.reshape(1, CC))
    return out
